# Initial kernel scaffold; baseline (speedup 1.0000x reference)
import jax
import jax.numpy as jnp
from jax.experimental import pallas as pl


def kernel(tokens, gate_ln_g, gate_ln_b, gate_w, gate_b, sal_ln_g, sal_ln_b, sal_w, sal_b, wq, wk, wv, wo, bq, bk, bv, bo, rel_ln_g, rel_ln_b):
    raise NotImplementedError("write your pallas kernel here")


# speedup vs baseline: 1.3532x; 1.3532x over previous
import functools

import jax
import jax.numpy as jnp
from jax import lax
from jax.experimental import pallas as pl
from jax.experimental.pallas import tpu as pltpu
from jax.experimental.pallas import tpu_sc as plsc

B, N, H = 8, 4096, 768
NUM_HEADS = 12
DH = H // NUM_HEADS
M = 64
EPS = 1e-5
BN = 512
NEG = -3.0e38


def _pass1_body(tok_ref, gw_ref, const_ref, sal_ref, pool_ref):
    j = pl.program_id(1)
    x = tok_ref[0]
    gw = gw_ref[0]
    gwsum = const_ref[0, 0]
    c = const_ref[0, 1]
    rowsum = jnp.sum(x, axis=1)
    rowsq = jnp.sum(x * x, axis=1)
    rowgw = jnp.sum(x * gw[None, :], axis=1)
    m = rowsum * (1.0 / H)
    v = rowsq * (1.0 / H) - m * m
    sal_ref[0, 0, 0] = (rowgw - m * gwsum) * lax.rsqrt(v + EPS) + c
    colsum = jnp.sum(x, axis=0)

    @pl.when(j == 0)
    def _():
        pool_ref[0, 0] = colsum

    @pl.when(j > 0)
    def _():
        pool_ref[0, 0] += colsum


_PASS1_SPEC = dict(
    grid=(B, N // BN),
    in_specs=[
        pl.BlockSpec((1, BN, H), lambda b, j: (b, j, 0)),
        pl.BlockSpec((1, H), lambda b, j: (0, 0)),
        pl.BlockSpec(memory_space=pltpu.SMEM),
    ],
    out_specs=[
        pl.BlockSpec((1, 1, 1, BN), lambda b, j: (b, j, 0, 0)),
        pl.BlockSpec((1, 1, H), lambda b, j: (b, 0, 0)),
    ],
    out_shape=[
        jax.ShapeDtypeStruct((B, N // BN, 1, BN), jnp.float32),
        jax.ShapeDtypeStruct((B, 1, H), jnp.float32),
    ],
)


def _select_body(sal_ref, pools_ref, gg_ref, gb_ref, gw_ref, consts_ref,
                 idx_ref, gate_ref, pooled_ref):
    pooled = pools_ref[:][:, 0, :] * (1.0 / N)
    pooled_ref[:] = pooled[:, None, :]
    m = jnp.mean(pooled, axis=1, keepdims=True)
    v = jnp.mean((pooled - m) ** 2, axis=1, keepdims=True)
    ln = (pooled - m) * lax.rsqrt(v + EPS) * gg_ref[0][None, :] + gb_ref[0][None, :]
    logit = jnp.sum(ln * gw_ref[0][None, :], axis=1, keepdims=True) + consts_ref[0, 0]
    gate_ref[:] = 1.0 / (1.0 + jnp.exp(-logit))

    s0 = sal_ref[:]
    iota_n = lax.broadcasted_iota(jnp.int32, (B, N), 1)
    iota_m = lax.broadcasted_iota(jnp.int32, (B, M), 1)

    def step(t, carry):
        s, acc = carry
        mx = jnp.max(s, axis=1, keepdims=True)
        ii = jnp.min(jnp.where(s == mx, iota_n, N), axis=1, keepdims=True)
        acc = jnp.where(iota_m == t, ii, acc)
        s = jnp.where(iota_n == ii, NEG, s)
        return s, acc

    _, acc = lax.fori_loop(0, M, step, (s0, jnp.zeros((B, M), jnp.int32)))
    idx_ref[:] = acc


_SELECT_SPEC = dict(
    in_specs=[
        pl.BlockSpec((B, N), lambda: (0, 0)),
        pl.BlockSpec((B, 1, H), lambda: (0, 0, 0)),
        pl.BlockSpec((1, H), lambda: (0, 0)),
        pl.BlockSpec((1, H), lambda: (0, 0)),
        pl.BlockSpec((1, H), lambda: (0, 0)),
        pl.BlockSpec(memory_space=pltpu.SMEM),
    ],
    out_specs=[
        pl.BlockSpec((B, M), lambda: (0, 0)),
        pl.BlockSpec((B, 1), lambda: (0, 0)),
        pl.BlockSpec((B, 1, H), lambda: (0, 0, 0)),
    ],
    out_shape=[
        jax.ShapeDtypeStruct((B, M), jnp.int32),
        jax.ShapeDtypeStruct((B, 1), jnp.float32),
        jax.ShapeDtypeStruct((B, 1, H), jnp.float32),
    ],
)


def _sc_gather(tokens_flat, flat_idx):
    info = plsc.get_sparse_core_info()
    nw = info.num_cores * info.num_subcores
    rows = B * M
    per_w = rows // nw
    mesh = plsc.VectorSubcoreMesh(core_axis_name="c", subcore_axis_name="s")

    @functools.partial(
        pl.kernel,
        mesh=mesh,
        out_type=jax.ShapeDtypeStruct((rows, H), jnp.float32),
        scratch_types=[
            pltpu.VMEM((per_w,), jnp.int32),
            pltpu.VMEM((per_w, H), jnp.float32),
            pltpu.SemaphoreType.DMA,
        ],
    )
    def gather(tok_hbm, idx_hbm, out_hbm, idx_v, rows_v, sem):
        wid = lax.axis_index("s") * info.num_cores + lax.axis_index("c")
        base = wid * per_w
        pltpu.sync_copy(idx_hbm.at[pl.ds(base, per_w)], idx_v)
        pltpu.async_copy(tok_hbm.at[idx_v], rows_v, sem).wait()
        pltpu.sync_copy(rows_v, out_hbm.at[pl.ds(base, per_w)])

    return gather(tokens_flat, flat_idx)


def _mha_body(sel_ref, wq_ref, wk_ref, wv_ref, wo_ref, bq_ref, bk_ref,
              bv_ref, bo_ref, rg_ref, rb_ref, gate_ref, pooled_ref,
              u_ref, p2_ref):
    b = pl.program_id(0)
    dn = (((1,), (1,)), ((), ()))
    sel = sel_ref[0]
    q = lax.dot_general(sel, wq_ref[:], dn, preferred_element_type=jnp.float32) + bq_ref[0][None, :]
    k = lax.dot_general(sel, wk_ref[:], dn, preferred_element_type=jnp.float32) + bk_ref[0][None, :]
    v = lax.dot_general(sel, wv_ref[:], dn, preferred_element_type=jnp.float32) + bv_ref[0][None, :]
    scale = 1.0 / (DH ** 0.5)
    outs = []
    for h in range(NUM_HEADS):
        sl = slice(h * DH, (h + 1) * DH)
        qh, kh, vh = q[:, sl], k[:, sl], v[:, sl]
        sc = lax.dot_general(qh, kh, dn, preferred_element_type=jnp.float32) * scale
        mx = jnp.max(sc, axis=1, keepdims=True)
        e = jnp.exp(sc - mx)
        p = e / jnp.sum(e, axis=1, keepdims=True)
        outs.append(jnp.dot(p, vh, preferred_element_type=jnp.float32))
    ao = jnp.concatenate(outs, axis=1)
    rel = lax.dot_general(ao, wo_ref[:], dn, preferred_element_type=jnp.float32) + bo_ref[0][None, :]
    y = sel + rel
    m = jnp.mean(y, axis=1, keepdims=True)
    var = jnp.mean((y - m) ** 2, axis=1, keepdims=True)
    ro = (y - m) * lax.rsqrt(var + EPS) * rg_ref[0][None, :] + rb_ref[0][None, :]
    g = gate_ref[b, 0]
    u_ref[0] = g * ro
    p2_ref[0, 0] = pooled_ref[0, 0] + (g / N) * jnp.sum(ro, axis=0)


_MHA_SPEC = dict(
    grid=(B,),
    in_specs=[
        pl.BlockSpec((1, M, H), lambda b: (b, 0, 0)),
        pl.BlockSpec((H, H), lambda b: (0, 0)),
        pl.BlockSpec((H, H), lambda b: (0, 0)),
        pl.BlockSpec((H, H), lambda b: (0, 0)),
        pl.BlockSpec((H, H), lambda b: (0, 0)),
        pl.BlockSpec((1, H), lambda b: (0, 0)),
        pl.BlockSpec((1, H), lambda b: (0, 0)),
        pl.BlockSpec((1, H), lambda b: (0, 0)),
        pl.BlockSpec((1, H), lambda b: (0, 0)),
        pl.BlockSpec((1, H), lambda b: (0, 0)),
        pl.BlockSpec((1, H), lambda b: (0, 0)),
        pl.BlockSpec(memory_space=pltpu.SMEM),
        pl.BlockSpec((1, 1, H), lambda b: (b, 0, 0)),
    ],
    out_specs=[
        pl.BlockSpec((1, M, H), lambda b: (b, 0, 0)),
        pl.BlockSpec((1, 1, H), lambda b: (b, 0, 0)),
    ],
    out_shape=[
        jax.ShapeDtypeStruct((B, M, H), jnp.float32),
        jax.ShapeDtypeStruct((B, 1, H), jnp.float32),
    ],
)


def _combine_body(tok_ref, u_ref, idx_ref, out_ref):
    j = pl.program_id(1)
    x = tok_ref[0]
    u = u_ref[0]
    ids = idx_ref[0, 0]
    rows = lax.broadcasted_iota(jnp.int32, (BN, M), 0) + j * BN
    onehot = (rows == ids[None, :]).astype(jnp.float32)
    out_ref[0] = x + jnp.dot(onehot, u, preferred_element_type=jnp.float32)


_COMBINE_SPEC = dict(
    grid=(B, N // BN),
    in_specs=[
        pl.BlockSpec((1, BN, H), lambda b, j: (b, j, 0)),
        pl.BlockSpec((1, M, H), lambda b, j: (b, 0, 0)),
        pl.BlockSpec((1, 1, M), lambda b, j: (b, 0, 0)),
    ],
    out_specs=pl.BlockSpec((1, BN, H), lambda b, j: (b, j, 0)),
    out_shape=jax.ShapeDtypeStruct((B, N, H), jnp.float32),
)


def kernel(tokens, gate_ln_g, gate_ln_b, gate_w, gate_b, sal_ln_g, sal_ln_b,
           sal_w, sal_b, wq, wk, wv, wo, bq, bk, bv, bo, rel_ln_g, rel_ln_b):
    f32 = jnp.float32
    gw = (sal_ln_g * sal_w[0]).reshape(1, H).astype(f32)
    sal_consts = jnp.stack(
        [jnp.sum(gw), sal_b[0] + jnp.dot(sal_ln_b, sal_w[0])]).reshape(1, 2)

    sal, pooled_sum = pl.pallas_call(_pass1_body, **_PASS1_SPEC)(
        tokens, gw, sal_consts)

    gate_consts = gate_b.reshape(1, 1)
    idx, gate, pooled = pl.pallas_call(_select_body, **_SELECT_SPEC)(
        sal.reshape(B, N), pooled_sum, gate_ln_g.reshape(1, H),
        gate_ln_b.reshape(1, H), gate_w, gate_consts)

    flat_idx = (idx + (jnp.arange(B, dtype=jnp.int32) * N)[:, None]).reshape(B * M)
    selected = _sc_gather(tokens.reshape(B * N, H), flat_idx)

    u, pooled2 = pl.pallas_call(_mha_body, **_MHA_SPEC)(
        selected.reshape(B, M, H), wq, wk, wv, wo,
        bq.reshape(1, H), bk.reshape(1, H), bv.reshape(1, H), bo.reshape(1, H),
        rel_ln_g.reshape(1, H), rel_ln_b.reshape(1, H), gate, pooled)

    tokens2 = pl.pallas_call(_combine_body, **_COMBINE_SPEC)(
        tokens, u, idx.reshape(B, 1, M))
    return pooled2.reshape(B, H), tokens2
